# baseline (speedup 1.0000x reference)
import numpy as np
import jax
import jax.numpy as jnp
from jax.experimental import pallas as pl
from jax.experimental.pallas import tpu as pltpu

_EPS = np.float32(1e-05)
_ROWS = 8192
_COLS = 1024
_BLOCK_ROWS = 256
_GRID = _ROWS // _BLOCK_ROWS
_CHUNK = 8
_NCHUNK = _BLOCK_ROWS // _CHUNK

_LOG2E = np.float32(1.4426950408889634)
_LN2 = np.float32(0.6931471805599453)
_LO = np.float32(-np.log(np.float32(1.0) - _EPS) / np.log(2.0))
_HI = np.float32(-np.log(_EPS) / np.log(2.0))
_THR2 = np.float32(-np.log2(9.0))


def _loss(x, a):
    y = x * _LOG2E
    zb = jax.lax.bitcast_convert_type(y, jnp.uint32) ^ (
        jax.lax.bitcast_convert_type(a, jnp.uint32) << 8
    )
    z = jax.lax.bitcast_convert_type(zb, jnp.float32)
    return jnp.minimum(jnp.maximum(jnp.log2(1.0 + jnp.exp2(z)), _LO), _HI), y


def _body(pred_ref, act_ref, out_ref, acc_ref, cnt_ref, row0_ref):
    i = pl.program_id(0)

    @pl.when(i == 0)
    def _init():
        acc_ref[...] = jnp.zeros_like(acc_ref)
        cnt_ref[...] = jnp.zeros_like(cnt_ref)
        row0_ref[...] = _loss(pred_ref[0:1, :], act_ref[0:1, :])[0]

    loss_acc = jnp.zeros((_CHUNK, _COLS), jnp.float32)
    a_acc = jnp.zeros((_CHUNK, _COLS), jnp.float32)
    thr_acc = jnp.zeros((_CHUNK, _COLS), jnp.float32)
    for c in range(_NCHUNK):
        sl = slice(c * _CHUNK, (c + 1) * _CHUNK)
        x = pred_ref[sl, :]
        a = act_ref[sl, :]
        l2, y = _loss(x, a)
        loss_acc = loss_acc + l2
        a_acc = a_acc + a
        thr_acc = thr_acc + jnp.where(y <= _THR2, 1.0, 0.0)

    acc_ref[...] += loss_acc
    cnt_ref[...] += thr_acc - a_acc

    @pl.when(i == _GRID - 1)
    def _finish():
        cnt_cols = jnp.sum(cnt_ref[...], axis=0, keepdims=True) + np.float32(_ROWS)
        mask = cnt_cols > 1.5
        corr = jnp.where(mask, row0_ref[...], 0.0)
        total = jnp.sum(acc_ref[...]) - jnp.sum(corr)
        out_ref[...] = jnp.reshape(total * (_LN2 / (_ROWS * _COLS)), (1, 1))


def kernel(pred, actual):
    p2 = pred.reshape(_ROWS, _COLS)
    a2 = actual.reshape(_ROWS, _COLS)
    res = pl.pallas_call(
        _body,
        grid=(_GRID,),
        in_specs=[
            pl.BlockSpec((_BLOCK_ROWS, _COLS), lambda i: (i, 0)),
            pl.BlockSpec((_BLOCK_ROWS, _COLS), lambda i: (i, 0)),
        ],
        out_specs=pl.BlockSpec((1, 1), lambda i: (0, 0)),
        out_shape=jax.ShapeDtypeStruct((1, 1), jnp.float32),
        scratch_shapes=[
            pltpu.VMEM((_CHUNK, _COLS), jnp.float32),
            pltpu.VMEM((_CHUNK, _COLS), jnp.float32),
            pltpu.VMEM((1, _COLS), jnp.float32),
        ],
        compiler_params=pltpu.CompilerParams(
            dimension_semantics=("arbitrary",),
        ),
    )(p2, a2)
    return res[0, 0]


# speedup vs baseline: 1.4014x; 1.4014x over previous
import numpy as np
import jax
import jax.numpy as jnp
from jax.experimental import pallas as pl
from jax.experimental.pallas import tpu as pltpu

_EPS = np.float32(1e-05)
_ROWS = 8192
_COLS = 1024
_BLOCK_ROWS = 1024
_GRID = _ROWS // _BLOCK_ROWS
_CHUNK = 8
_NCHUNK = _BLOCK_ROWS // _CHUNK

_LOG2E = np.float32(1.4426950408889634)
_LN2 = np.float32(0.6931471805599453)
_LO = np.float32(-np.log(np.float32(1.0) - _EPS) / np.log(2.0))
_HI = np.float32(-np.log(_EPS) / np.log(2.0))
_THR2 = np.float32(-np.log2(9.0))


def _loss(x, a):
    y = x * _LOG2E
    zb = jax.lax.bitcast_convert_type(y, jnp.uint32) ^ (
        jax.lax.bitcast_convert_type(a, jnp.uint32) << 8
    )
    z = jax.lax.bitcast_convert_type(zb, jnp.float32)
    return jnp.minimum(jnp.maximum(jnp.log2(1.0 + jnp.exp2(z)), _LO), _HI), y


def _body(pred_ref, act_ref, out_ref, acc_ref, cnt_ref, row0_ref):
    i = pl.program_id(0)

    @pl.when(i == 0)
    def _init():
        acc_ref[...] = jnp.zeros_like(acc_ref)
        cnt_ref[...] = jnp.zeros_like(cnt_ref)
        row0_ref[...] = _loss(pred_ref[0:1, :], act_ref[0:1, :])[0]

    _NACC = 4
    zero = jnp.zeros((_CHUNK, _COLS), jnp.float32)
    loss_acc = [zero] * _NACC
    a_acc = [zero] * _NACC
    thr_acc = [zero] * _NACC
    for c in range(_NCHUNK):
        sl = slice(c * _CHUNK, (c + 1) * _CHUNK)
        x = pred_ref[sl, :]
        a = act_ref[sl, :]
        l2, y = _loss(x, a)
        k = c % _NACC
        loss_acc[k] = loss_acc[k] + l2
        a_acc[k] = a_acc[k] + a
        thr_acc[k] = thr_acc[k] + jnp.where(y <= _THR2, 1.0, 0.0)

    acc_ref[...] += (loss_acc[0] + loss_acc[1]) + (loss_acc[2] + loss_acc[3])
    cnt_ref[...] += ((thr_acc[0] + thr_acc[1]) + (thr_acc[2] + thr_acc[3])) - (
        (a_acc[0] + a_acc[1]) + (a_acc[2] + a_acc[3])
    )

    @pl.when(i == _GRID - 1)
    def _finish():
        cnt_cols = jnp.sum(cnt_ref[...], axis=0, keepdims=True) + np.float32(_ROWS)
        mask = cnt_cols > 1.5
        corr = jnp.where(mask, row0_ref[...], 0.0)
        total = jnp.sum(acc_ref[...]) - jnp.sum(corr)
        out_ref[...] = jnp.reshape(total * (_LN2 / (_ROWS * _COLS)), (1, 1))


def kernel(pred, actual):
    p2 = pred.reshape(_ROWS, _COLS)
    a2 = actual.reshape(_ROWS, _COLS)
    res = pl.pallas_call(
        _body,
        grid=(_GRID,),
        in_specs=[
            pl.BlockSpec((_BLOCK_ROWS, _COLS), lambda i: (i, 0)),
            pl.BlockSpec((_BLOCK_ROWS, _COLS), lambda i: (i, 0)),
        ],
        out_specs=pl.BlockSpec((1, 1), lambda i: (0, 0)),
        out_shape=jax.ShapeDtypeStruct((1, 1), jnp.float32),
        scratch_shapes=[
            pltpu.VMEM((_CHUNK, _COLS), jnp.float32),
            pltpu.VMEM((_CHUNK, _COLS), jnp.float32),
            pltpu.VMEM((1, _COLS), jnp.float32),
        ],
        compiler_params=pltpu.CompilerParams(
            dimension_semantics=("arbitrary",),
        ),
    )(p2, a2)
    return res[0, 0]
